# Initial kernel scaffold; baseline (speedup 1.0000x reference)
import jax
import jax.numpy as jnp
from jax.experimental import pallas as pl


def kernel(node_features, edge_index, etype, rel_emb_0, rel_emb_1):
    raise NotImplementedError("write your pallas kernel here")


# speedup vs baseline: 3.8792x; 3.8792x over previous
import functools
import jax
import jax.numpy as jnp
from jax import lax
from jax.experimental import pallas as pl
from jax.experimental.pallas import tpu as pltpu, tpu_sc as plsc

N_NODES = 10000
N_EDGES = 320000
D = 128
N_REL = 100

NC = 2
NS = 16
NW = NC * NS
E_PER_W = N_EDGES // NW
CHUNK = 80
N_CHUNKS = E_PER_W // CHUNK
ROWS_MAIN = 624

_mesh = plsc.VectorSubcoreMesh(
    core_axis_name="c", subcore_axis_name="s", num_cores=NC, num_subcores=NS)


def _layer_body(h, rel, src, dst, et, out0, out1, idx_s, idx_e, idx_d, rows,
                rel_rows, acc, rel_sp):
  cid = lax.axis_index("c")
  sid = lax.axis_index("s")

  @pl.when(sid == 0)
  def _():
    pltpu.sync_copy(rel, rel_sp)

  zv = jnp.zeros((16,), jnp.float32)

  def _zero_rows(r, _):
    for c in range(8):
      rows[r, pl.ds(c * 16, 16)] = zv
    return _

  lax.fori_loop(0, CHUNK, _zero_rows, None)
  row0 = sid * ROWS_MAIN
  for k in range(ROWS_MAIN // CHUNK):
    pltpu.sync_copy(rows, acc.at[pl.ds(row0 + k * CHUNK, CHUNK)])
  rem = ROWS_MAIN % CHUNK
  pltpu.sync_copy(rows.at[pl.ds(0, rem)],
                  acc.at[pl.ds(row0 + ROWS_MAIN - rem, rem)])

  @pl.when(sid == NS - 1)
  def _():
    pltpu.sync_copy(rows.at[pl.ds(0, N_NODES - NS * ROWS_MAIN)],
                    acc.at[pl.ds(NS * ROWS_MAIN, N_NODES - NS * ROWS_MAIN)])

  plsc.subcore_barrier()

  wid = cid * NS + sid
  base_e = wid * E_PER_W

  def _chunk(i, _):
    off = base_e + i * CHUNK
    pltpu.sync_copy(src.at[pl.ds(off, CHUNK)], idx_s)
    pltpu.sync_copy(et.at[pl.ds(off, CHUNK)], idx_e)
    pltpu.sync_copy(dst.at[pl.ds(off, CHUNK)], idx_d)
    pltpu.sync_copy(h.at[idx_s], rows)
    pltpu.sync_copy(rel_sp.at[idx_e], rel_rows)

    def _mul(r, carry):
      for c in range(8):
        s = pl.ds(c * 16, 16)
        rows[r, s] = rows[r, s] * rel_rows[r, s]
      return carry

    lax.fori_loop(0, CHUNK, _mul, None)
    pltpu.sync_copy(rows, acc.at[idx_d], add=True)
    return _

  lax.fori_loop(0, N_CHUNKS, _chunk, None)

  plsc.subcore_barrier()
  for c, out in ((0, out0), (1, out1)):

    @pl.when(cid == c)
    def _():
      pltpu.sync_copy(acc.at[pl.ds(row0, ROWS_MAIN)],
                      out.at[pl.ds(row0, ROWS_MAIN)])

      @pl.when(sid == NS - 1)
      def _():
        tail = N_NODES - NS * ROWS_MAIN
        pltpu.sync_copy(acc.at[pl.ds(NS * ROWS_MAIN, tail)],
                        out.at[pl.ds(NS * ROWS_MAIN, tail)])


_layer = pl.kernel(
    _layer_body,
    out_type=(jax.ShapeDtypeStruct((N_NODES, D), jnp.float32),
              jax.ShapeDtypeStruct((N_NODES, D), jnp.float32)),
    mesh=_mesh,
    scratch_types=[
        pltpu.VMEM((CHUNK,), jnp.int32),
        pltpu.VMEM((CHUNK,), jnp.int32),
        pltpu.VMEM((CHUNK,), jnp.int32),
        pltpu.VMEM((CHUNK, D), jnp.float32),
        pltpu.VMEM((CHUNK, D), jnp.float32),
        pltpu.VMEM_SHARED((N_NODES, D), jnp.float32),
        pltpu.VMEM_SHARED((N_REL, D), jnp.float32),
    ],
)

_FLAT = N_NODES * D
_F_PER_W = _FLAT // NW
_CCHUNK = 8000
_N_CCHUNKS = _F_PER_W // _CCHUNK


def _combine_body(p0, p1, out, a, b):
  cid = lax.axis_index("c")
  sid = lax.axis_index("s")
  base = (cid * NS + sid) * _F_PER_W

  def _chunk(k, _):
    off = base + k * _CCHUNK
    pltpu.sync_copy(p0.at[pl.ds(off, _CCHUNK)], a)
    pltpu.sync_copy(p1.at[pl.ds(off, _CCHUNK)], b)

    def _add(j, carry):
      s = pl.ds(j * 16, 16)
      a[s] = a[s] + b[s]
      return carry

    lax.fori_loop(0, _CCHUNK // 16, _add, None)
    pltpu.sync_copy(a, out.at[pl.ds(off, _CCHUNK)])
    return _

  lax.fori_loop(0, _N_CCHUNKS, _chunk, None)


_combine = pl.kernel(
    _combine_body,
    out_type=jax.ShapeDtypeStruct((_FLAT,), jnp.float32),
    mesh=_mesh,
    scratch_types=[
        pltpu.VMEM((_CCHUNK,), jnp.float32),
        pltpu.VMEM((_CCHUNK,), jnp.float32),
    ],
)


def kernel(node_features, edge_index, etype, rel_emb_0, rel_emb_1):
  src = edge_index[0].astype(jnp.int32)
  dst = edge_index[1].astype(jnp.int32)
  et = etype.astype(jnp.int32)
  h = node_features
  for rel in (rel_emb_0, rel_emb_1):
    p0, p1 = _layer(h, rel, src, dst, et)
    h = _combine(p0.reshape(_FLAT), p1.reshape(_FLAT)).reshape(N_NODES, D)
  return h


# speedup vs baseline: 6.1623x; 1.5885x over previous
import jax
import jax.numpy as jnp
from jax import lax
from jax.experimental import pallas as pl
from jax.experimental.pallas import tpu as pltpu, tpu_sc as plsc

N_NODES = 10000
N_EDGES = 320000
D = 128
N_REL = 100

NC = 2
NS = 16
NW = NC * NS
E_PER_W = N_EDGES // NW
CHUNK = 40
N_CHUNKS = E_PER_W // CHUNK
ROWS_MAIN = 624

_mesh = plsc.VectorSubcoreMesh(
    core_axis_name="c", subcore_axis_name="s", num_cores=NC, num_subcores=NS)


def _layer_body(h, rel, src1, dst1, et1, out0, out1,
                idx_s, idx_e, idx_d0, idx_d1, rows0, rows1, relr0, relr1,
                acc, rel_sp,
                sem_gh0, sem_gh1, sem_gr0, sem_gr1, sem_s0, sem_s1,
                sem_d0, sem_d1):
  cid = lax.axis_index("c")
  sid = lax.axis_index("s")

  @pl.when(sid == 0)
  def _():
    pltpu.sync_copy(rel, rel_sp)

  zv = jnp.zeros((16,), jnp.float32)

  @plsc.parallel_loop(0, CHUNK)
  def _(r):
    for c in range(8):
      rows0[r, pl.ds(c * 16, 16)] = zv

  row0 = sid * ROWS_MAIN
  for k in range(ROWS_MAIN // CHUNK):
    pltpu.sync_copy(rows0, acc.at[pl.ds(row0 + k * CHUNK, CHUNK)])
  rem = ROWS_MAIN % CHUNK
  pltpu.sync_copy(rows0.at[pl.ds(0, rem)],
                  acc.at[pl.ds(row0 + ROWS_MAIN - rem, rem)])

  @pl.when(sid == NS - 1)
  def _():
    tail = N_NODES - NS * ROWS_MAIN
    pltpu.sync_copy(rows0.at[pl.ds(0, tail)],
                    acc.at[pl.ds(NS * ROWS_MAIN, tail)])

  plsc.subcore_barrier()

  wid = cid * NS + sid
  base_e = wid * E_PER_W
  pltpu.sync_copy(src1.at[pl.ds(base_e, E_PER_W)], idx_s)
  pltpu.sync_copy(et1.at[pl.ds(base_e, E_PER_W)], idx_e)

  bufs = ((rows0, relr0, idx_d0, sem_gh0, sem_gr0, sem_s0, sem_d0),
          (rows1, relr1, idx_d1, sem_gh1, sem_gr1, sem_s1, sem_d1))

  def issue_gather(i, b):
    rows, relr, _, sem_gh, sem_gr, _, _ = bufs[b]
    pltpu.async_copy(h.at[idx_s.at[pl.ds(i * CHUNK, CHUNK)]], rows, sem_gh)
    pltpu.async_copy(rel_sp.at[idx_e.at[pl.ds(i * CHUNK, CHUNK)]], relr,
                     sem_gr)

  def wait_gather(i, b):
    rows, relr, _, sem_gh, sem_gr, _, _ = bufs[b]
    pltpu.make_async_copy(h.at[idx_s.at[pl.ds(i * CHUNK, CHUNK)]], rows,
                          sem_gh).wait()
    pltpu.make_async_copy(rel_sp.at[idx_e.at[pl.ds(i * CHUNK, CHUNK)]], relr,
                          sem_gr).wait()

  def refill_d(i, b):
    idx_d, sem_d = bufs[b][2], bufs[b][6]
    pltpu.async_copy(dst1.at[pl.ds(base_e + i * CHUNK, CHUNK)], idx_d, sem_d)

  def wait_refill_d(i, b):
    idx_d, sem_d = bufs[b][2], bufs[b][6]
    pltpu.make_async_copy(dst1.at[pl.ds(base_e + i * CHUNK, CHUNK)], idx_d,
                          sem_d).wait()

  def mul(b):
    rows, relr = bufs[b][0], bufs[b][1]

    @plsc.parallel_loop(0, CHUNK, unroll=4)
    def _(r):
      for c in range(8):
        s = pl.ds(c * 16, 16)
        rows[r, s] = rows[r, s] * relr[r, s]

  def issue_scatter(i, b):
    rows, idx_d, sem_s = bufs[b][0], bufs[b][2], bufs[b][5]
    pltpu.async_copy(rows, acc.at[idx_d], sem_s, add=True)

  def wait_scatter(b):
    rows, idx_d, sem_s = bufs[b][0], bufs[b][2], bufs[b][5]
    pltpu.make_async_copy(rows, acc.at[idx_d], sem_s).wait()

  refill_d(0, 0)
  refill_d(1, 1)
  issue_gather(0, 0)
  n_pairs = N_CHUNKS // 2

  def _pair(g, _):
    i0 = 2 * g
    wait_gather(i0, 0)
    mul(0)
    wait_refill_d(i0, 0)
    issue_scatter(i0, 0)

    @pl.when(g > 0)
    def _():
      wait_scatter(1)
      refill_d(i0 + 1, 1)

    issue_gather(i0 + 1, 1)
    wait_gather(i0 + 1, 1)
    mul(1)
    wait_refill_d(i0 + 1, 1)
    issue_scatter(i0 + 1, 1)
    wait_scatter(0)

    @pl.when(g < n_pairs - 1)
    def _():
      refill_d(i0 + 2, 0)
      issue_gather(i0 + 2, 0)

    return _

  lax.fori_loop(0, n_pairs, _pair, None)
  wait_scatter(1)

  plsc.subcore_barrier()
  for c, out in ((0, out0), (1, out1)):

    @pl.when(cid == c)
    def _():
      pltpu.sync_copy(acc.at[pl.ds(row0, ROWS_MAIN)],
                      out.at[pl.ds(row0, ROWS_MAIN)])

      @pl.when(sid == NS - 1)
      def _():
        tail = N_NODES - NS * ROWS_MAIN
        pltpu.sync_copy(acc.at[pl.ds(NS * ROWS_MAIN, tail)],
                        out.at[pl.ds(NS * ROWS_MAIN, tail)])


_layer = pl.kernel(
    _layer_body,
    out_type=(jax.ShapeDtypeStruct((N_NODES, D), jnp.float32),
              jax.ShapeDtypeStruct((N_NODES, D), jnp.float32)),
    mesh=_mesh,
    scratch_types=[
        pltpu.VMEM((E_PER_W,), jnp.int32),
        pltpu.VMEM((E_PER_W,), jnp.int32),
        pltpu.VMEM((CHUNK,), jnp.int32),
        pltpu.VMEM((CHUNK,), jnp.int32),
        pltpu.VMEM((CHUNK, D), jnp.float32),
        pltpu.VMEM((CHUNK, D), jnp.float32),
        pltpu.VMEM((CHUNK, D), jnp.float32),
        pltpu.VMEM((CHUNK, D), jnp.float32),
        pltpu.VMEM_SHARED((N_NODES, D), jnp.float32),
        pltpu.VMEM_SHARED((N_REL, D), jnp.float32),
        pltpu.SemaphoreType.DMA,
        pltpu.SemaphoreType.DMA,
        pltpu.SemaphoreType.DMA,
        pltpu.SemaphoreType.DMA,
        pltpu.SemaphoreType.DMA,
        pltpu.SemaphoreType.DMA,
        pltpu.SemaphoreType.DMA,
        pltpu.SemaphoreType.DMA,
    ],
)

_FLAT = N_NODES * D
_F_PER_W = _FLAT // NW
_CCHUNK = 8000
_N_CCHUNKS = _F_PER_W // _CCHUNK


def _combine_body(p0, p1, out, a, b):
  cid = lax.axis_index("c")
  sid = lax.axis_index("s")
  base = (cid * NS + sid) * _F_PER_W

  def _chunk(k, _):
    off = base + k * _CCHUNK
    pltpu.sync_copy(p0.at[pl.ds(off, _CCHUNK)], a)
    pltpu.sync_copy(p1.at[pl.ds(off, _CCHUNK)], b)

    @plsc.parallel_loop(0, _CCHUNK // 16, unroll=4)
    def _(j):
      s = pl.ds(j * 16, 16)
      a[s] = a[s] + b[s]

    pltpu.sync_copy(a, out.at[pl.ds(off, _CCHUNK)])
    return _

  lax.fori_loop(0, _N_CCHUNKS, _chunk, None)


_combine = pl.kernel(
    _combine_body,
    out_type=jax.ShapeDtypeStruct((_FLAT,), jnp.float32),
    mesh=_mesh,
    scratch_types=[
        pltpu.VMEM((_CCHUNK,), jnp.float32),
        pltpu.VMEM((_CCHUNK,), jnp.float32),
    ],
)


def kernel(node_features, edge_index, etype, rel_emb_0, rel_emb_1):
  src = edge_index[0].astype(jnp.int32)
  dst = edge_index[1].astype(jnp.int32)
  et = etype.astype(jnp.int32)
  h = node_features
  for rel in (rel_emb_0, rel_emb_1):
    p0, p1 = _layer(h, rel, src, dst, et)
    h = _combine(p0.reshape(_FLAT), p1.reshape(_FLAT)).reshape(N_NODES, D)
  return h


# speedup vs baseline: 10.3817x; 1.6847x over previous
import jax
import jax.numpy as jnp
from jax import lax
from jax.experimental import pallas as pl
from jax.experimental.pallas import tpu as pltpu, tpu_sc as plsc

N_NODES = 10000
N_EDGES = 320000
D = 128
N_REL = 100

NC = 2
NS = 16
NW = NC * NS
E_PER_W = N_EDGES // NW
CHUNK = 40
N_CHUNKS = E_PER_W // CHUNK
ROWS_MAIN = 624

_mesh = plsc.VectorSubcoreMesh(
    core_axis_name="c", subcore_axis_name="s", num_cores=NC, num_subcores=NS)


def _layer_body(h, rel, src1, dst1, et1, out0, out1,
                rows0, rows1, rows2, relr0, relr1, relr2,
                sb0, sb1, sb2, eb0, eb1, eb2, db0, db1, db2,
                acc, rel_sp,
                sgh0, sgh1, sgh2, sgr0, sgr1, sgr2,
                sse0, sse1, sse2, sd0, sd1, sd2, ss0, ss1, ss2):
  cid = lax.axis_index("c")
  sid = lax.axis_index("s")

  @pl.when(sid == 0)
  def _():
    pltpu.sync_copy(rel, rel_sp)

  zv = jnp.zeros((16,), jnp.float32)

  @plsc.parallel_loop(0, CHUNK)
  def _(r):
    for c in range(8):
      rows0[r, pl.ds(c * 16, 16)] = zv

  row0 = sid * ROWS_MAIN
  for k in range(ROWS_MAIN // CHUNK):
    pltpu.sync_copy(rows0, acc.at[pl.ds(row0 + k * CHUNK, CHUNK)])
  rem = ROWS_MAIN % CHUNK
  pltpu.sync_copy(rows0.at[pl.ds(0, rem)],
                  acc.at[pl.ds(row0 + ROWS_MAIN - rem, rem)])

  @pl.when(sid == NS - 1)
  def _():
    tail = N_NODES - NS * ROWS_MAIN
    pltpu.sync_copy(rows0.at[pl.ds(0, tail)],
                    acc.at[pl.ds(NS * ROWS_MAIN, tail)])

  plsc.subcore_barrier()

  wid = cid * NS + sid
  base_e = wid * E_PER_W

  bufs = ((rows0, relr0, sb0, eb0, db0, sgh0, sgr0, sse0, sd0, ss0),
          (rows1, relr1, sb1, eb1, db1, sgh1, sgr1, sse1, sd1, ss1),
          (rows2, relr2, sb2, eb2, db2, sgh2, sgr2, sse2, sd2, ss2))

  def refill_se(i, b):
    _, _, sb, eb, _, _, _, sse, _, _ = bufs[b]
    pltpu.async_copy(src1.at[pl.ds(base_e + i * CHUNK, CHUNK)], sb, sse)
    pltpu.async_copy(et1.at[pl.ds(base_e + i * CHUNK, CHUNK)], eb, sse)

  def wait_se(i, b):
    _, _, sb, eb, _, _, _, sse, _, _ = bufs[b]
    pltpu.make_async_copy(src1.at[pl.ds(base_e + i * CHUNK, CHUNK)], sb,
                          sse).wait()
    pltpu.make_async_copy(et1.at[pl.ds(base_e + i * CHUNK, CHUNK)], eb,
                          sse).wait()

  def refill_d(i, b):
    db, sd = bufs[b][4], bufs[b][8]
    pltpu.async_copy(dst1.at[pl.ds(base_e + i * CHUNK, CHUNK)], db, sd)

  def wait_d(i, b):
    db, sd = bufs[b][4], bufs[b][8]
    pltpu.make_async_copy(dst1.at[pl.ds(base_e + i * CHUNK, CHUNK)], db,
                          sd).wait()

  def issue_gather(b):
    rows, relr, sb, eb = bufs[b][0], bufs[b][1], bufs[b][2], bufs[b][3]
    sgh, sgr = bufs[b][5], bufs[b][6]
    pltpu.async_copy(h.at[sb], rows, sgh)
    pltpu.async_copy(rel_sp.at[eb], relr, sgr)

  def wait_gather(b):
    rows, relr, sb, eb = bufs[b][0], bufs[b][1], bufs[b][2], bufs[b][3]
    sgh, sgr = bufs[b][5], bufs[b][6]
    pltpu.make_async_copy(h.at[sb], rows, sgh).wait()
    pltpu.make_async_copy(rel_sp.at[eb], relr, sgr).wait()

  def mul(b):
    rows, relr = bufs[b][0], bufs[b][1]

    @plsc.parallel_loop(0, CHUNK, unroll=4)
    def _(r):
      for c in range(8):
        s = pl.ds(c * 16, 16)
        rows[r, s] = rows[r, s] * relr[r, s]

  def issue_scatter(b):
    rows, db, ss = bufs[b][0], bufs[b][4], bufs[b][9]
    pltpu.async_copy(rows, acc.at[db], ss, add=True)

  def wait_scatter(b):
    rows, db, ss = bufs[b][0], bufs[b][4], bufs[b][9]
    pltpu.make_async_copy(rows, acc.at[db], ss).wait()

  def section(i, b, bn, bnn):
    @pl.when(i >= 2)
    def _():
      wait_scatter(bn)

    @pl.when(i + 1 < N_CHUNKS)
    def _():
      refill_d(i + 1, bn)
      wait_se(i + 1, bn)
      issue_gather(bn)

    @pl.when(i + 2 < N_CHUNKS)
    def _():
      refill_se(i + 2, bnn)

    wait_gather(b)
    mul(b)
    wait_d(i, b)
    issue_scatter(b)

  refill_se(0, 0)
  refill_se(1, 1)
  refill_d(0, 0)
  wait_se(0, 0)
  issue_gather(0)

  def _trio(g, _):
    for k in range(3):
      section(3 * g + k, k, (k + 1) % 3, (k + 2) % 3)
    return _

  n_trios = (N_CHUNKS - 1) // 3
  lax.fori_loop(0, n_trios, _trio, None)
  section(jnp.int32(N_CHUNKS - 1), (N_CHUNKS - 1) % 3, N_CHUNKS % 3,
          (N_CHUNKS + 1) % 3)
  wait_scatter((N_CHUNKS - 2) % 3)
  wait_scatter((N_CHUNKS - 1) % 3)

  plsc.subcore_barrier()
  for c, out in ((0, out0), (1, out1)):

    @pl.when(cid == c)
    def _():
      pltpu.sync_copy(acc.at[pl.ds(row0, ROWS_MAIN)],
                      out.at[pl.ds(row0, ROWS_MAIN)])

      @pl.when(sid == NS - 1)
      def _():
        tail = N_NODES - NS * ROWS_MAIN
        pltpu.sync_copy(acc.at[pl.ds(NS * ROWS_MAIN, tail)],
                        out.at[pl.ds(NS * ROWS_MAIN, tail)])


_layer = pl.kernel(
    _layer_body,
    out_type=(jax.ShapeDtypeStruct((N_NODES, D), jnp.float32),
              jax.ShapeDtypeStruct((N_NODES, D), jnp.float32)),
    mesh=_mesh,
    scratch_types=(
        [pltpu.VMEM((CHUNK, D), jnp.float32)] * 6
        + [pltpu.VMEM((CHUNK,), jnp.int32)] * 9
        + [pltpu.VMEM_SHARED((N_NODES, D), jnp.float32),
           pltpu.VMEM_SHARED((N_REL, D), jnp.float32)]
        + [pltpu.SemaphoreType.DMA] * 15
    ),
)

_FLAT = N_NODES * D
_F_PER_W = _FLAT // NW
_CCHUNK = 8000
_N_CCHUNKS = _F_PER_W // _CCHUNK


def _combine_body(p0, p1, out, a, b):
  cid = lax.axis_index("c")
  sid = lax.axis_index("s")
  base = (cid * NS + sid) * _F_PER_W

  def _chunk(k, _):
    off = base + k * _CCHUNK
    pltpu.sync_copy(p0.at[pl.ds(off, _CCHUNK)], a)
    pltpu.sync_copy(p1.at[pl.ds(off, _CCHUNK)], b)

    @plsc.parallel_loop(0, _CCHUNK // 16, unroll=4)
    def _(j):
      s = pl.ds(j * 16, 16)
      a[s] = a[s] + b[s]

    pltpu.sync_copy(a, out.at[pl.ds(off, _CCHUNK)])
    return _

  lax.fori_loop(0, _N_CCHUNKS, _chunk, None)


_combine = pl.kernel(
    _combine_body,
    out_type=jax.ShapeDtypeStruct((_FLAT,), jnp.float32),
    mesh=_mesh,
    scratch_types=[
        pltpu.VMEM((_CCHUNK,), jnp.float32),
        pltpu.VMEM((_CCHUNK,), jnp.float32),
    ],
)


def kernel(node_features, edge_index, etype, rel_emb_0, rel_emb_1):
  src = edge_index[0].astype(jnp.int32)
  dst = edge_index[1].astype(jnp.int32)
  et = etype.astype(jnp.int32)
  h = node_features
  for rel in (rel_emb_0, rel_emb_1):
    p0, p1 = _layer(h, rel, src, dst, et)
    h = _combine(p0.reshape(_FLAT), p1.reshape(_FLAT)).reshape(N_NODES, D)
  return h


# speedup vs baseline: 10.6975x; 1.0304x over previous
import jax
import jax.numpy as jnp
from jax import lax
from jax.experimental import pallas as pl
from jax.experimental.pallas import tpu as pltpu, tpu_sc as plsc

N_NODES = 10000
N_EDGES = 320000
D = 128
DH = D // 2
N_REL = 100

NC = 2
NS = 16
E_PER_T = N_EDGES // NS
CHUNK = 80
N_CHUNKS = E_PER_T // CHUNK
ROWS_MAIN = 624

_mesh = plsc.VectorSubcoreMesh(
    core_axis_name="c", subcore_axis_name="s", num_cores=NC, num_subcores=NS)


def _gnn_body(ha, hb, ra0, rb0, ra1, rb1, src1, dst1, et1, outa, outb,
              rows0, rows1, rows2, relr0, relr1, relr2,
              sb0, sb1, sb2, eb0, eb1, eb2, db0, db1, db2,
              acc1, acc2, rel_sp0, rel_sp1,
              sgh0, sgh1, sgh2, sgr0, sgr1, sgr2,
              sse0, sse1, sse2, sd0, sd1, sd2, ss0, ss1, ss2):
  cid = lax.axis_index("c")
  sid = lax.axis_index("s")

  @pl.when(jnp.logical_and(cid == 0, sid == 0))
  def _():
    pltpu.sync_copy(ra0, rel_sp0)
    pltpu.sync_copy(ra1, rel_sp1)

  @pl.when(jnp.logical_and(cid == 1, sid == 0))
  def _():
    pltpu.sync_copy(rb0, rel_sp0)
    pltpu.sync_copy(rb1, rel_sp1)

  zv = jnp.zeros((16,), jnp.float32)

  @plsc.parallel_loop(0, CHUNK)
  def _(r):
    for c in range(DH // 16):
      rows0[r, pl.ds(c * 16, 16)] = zv

  row0 = sid * ROWS_MAIN
  tail = N_NODES - NS * ROWS_MAIN
  for acc in (acc1, acc2):
    for k in range(ROWS_MAIN // CHUNK):
      pltpu.sync_copy(rows0, acc.at[pl.ds(row0 + k * CHUNK, CHUNK)])
    rem = ROWS_MAIN % CHUNK
    pltpu.sync_copy(rows0.at[pl.ds(0, rem)],
                    acc.at[pl.ds(row0 + ROWS_MAIN - rem, rem)])

    @pl.when(sid == NS - 1)
    def _():
      pltpu.sync_copy(rows0.at[pl.ds(0, tail)],
                      acc.at[pl.ds(NS * ROWS_MAIN, tail)])

  plsc.subcore_barrier()

  base_e = sid * E_PER_T

  bufs = ((rows0, relr0, sb0, eb0, db0, sgh0, sgr0, sse0, sd0, ss0),
          (rows1, relr1, sb1, eb1, db1, sgh1, sgr1, sse1, sd1, ss1),
          (rows2, relr2, sb2, eb2, db2, sgh2, sgr2, sse2, sd2, ss2))

  def refill_se(i, b):
    sb, eb, sse = bufs[b][2], bufs[b][3], bufs[b][7]
    pltpu.async_copy(src1.at[pl.ds(base_e + i * CHUNK, CHUNK)], sb, sse)
    pltpu.async_copy(et1.at[pl.ds(base_e + i * CHUNK, CHUNK)], eb, sse)

  def wait_se(i, b):
    sb, eb, sse = bufs[b][2], bufs[b][3], bufs[b][7]
    pltpu.make_async_copy(src1.at[pl.ds(base_e + i * CHUNK, CHUNK)], sb,
                          sse).wait()
    pltpu.make_async_copy(et1.at[pl.ds(base_e + i * CHUNK, CHUNK)], eb,
                          sse).wait()

  def refill_d(i, b):
    db, sd = bufs[b][4], bufs[b][8]
    pltpu.async_copy(dst1.at[pl.ds(base_e + i * CHUNK, CHUNK)], db, sd)

  def wait_d(i, b):
    db, sd = bufs[b][4], bufs[b][8]
    pltpu.make_async_copy(dst1.at[pl.ds(base_e + i * CHUNK, CHUNK)], db,
                          sd).wait()

  def mul(b):
    rows, relr = bufs[b][0], bufs[b][1]

    @plsc.parallel_loop(0, CHUNK, unroll=4)
    def _(r):
      for c in range(DH // 16):
        s = pl.ds(c * 16, 16)
        rows[r, s] = rows[r, s] * relr[r, s]

  def make_phase(src_tab, rel_sp, acc):
    per_core = isinstance(src_tab, tuple)

    def issue_gather(b):
      rows, relr = bufs[b][0], bufs[b][1]
      sb, eb = bufs[b][2], bufs[b][3]
      sgh, sgr = bufs[b][5], bufs[b][6]
      if per_core:
        @pl.when(cid == 0)
        def _():
          pltpu.async_copy(src_tab[0].at[sb], rows, sgh)

        @pl.when(cid == 1)
        def _():
          pltpu.async_copy(src_tab[1].at[sb], rows, sgh)
      else:
        pltpu.async_copy(src_tab.at[sb], rows, sgh)
      pltpu.async_copy(rel_sp.at[eb], relr, sgr)

    def wait_gather(b):
      rows, relr = bufs[b][0], bufs[b][1]
      sb, eb = bufs[b][2], bufs[b][3]
      sgh, sgr = bufs[b][5], bufs[b][6]
      ref = src_tab[0] if per_core else src_tab
      pltpu.make_async_copy(ref.at[sb], rows, sgh).wait()
      pltpu.make_async_copy(rel_sp.at[eb], relr, sgr).wait()

    def issue_scatter(b):
      rows, db, ss = bufs[b][0], bufs[b][4], bufs[b][9]
      pltpu.async_copy(rows, acc.at[db], ss, add=True)

    def wait_scatter(b):
      rows, db, ss = bufs[b][0], bufs[b][4], bufs[b][9]
      pltpu.make_async_copy(rows, acc.at[db], ss).wait()

    def section(i, b, bn, bnn):
      @pl.when(i >= 2)
      def _():
        wait_scatter(bn)

      @pl.when(i + 1 < N_CHUNKS)
      def _():
        refill_d(i + 1, bn)
        wait_se(i + 1, bn)
        issue_gather(bn)

      @pl.when(i + 2 < N_CHUNKS)
      def _():
        refill_se(i + 2, bnn)

      wait_gather(b)
      mul(b)
      wait_d(i, b)
      issue_scatter(b)

    def run():
      refill_se(0, 0)
      refill_se(1, 1)
      refill_d(0, 0)
      wait_se(0, 0)
      issue_gather(0)

      def _trio(g, _):
        for k in range(3):
          section(3 * g + k, k, (k + 1) % 3, (k + 2) % 3)
        return _

      n_trios = N_CHUNKS // 3
      lax.fori_loop(0, n_trios, _trio, None)
      for i in range(3 * n_trios, N_CHUNKS):
        section(jnp.int32(i), i % 3, (i + 1) % 3, (i + 2) % 3)
      wait_scatter((N_CHUNKS - 2) % 3)
      wait_scatter((N_CHUNKS - 1) % 3)

    return run

  make_phase((ha, hb), rel_sp0, acc1)()
  plsc.subcore_barrier()
  make_phase(acc1, rel_sp1, acc2)()
  plsc.subcore_barrier()

  for c, out in ((0, outa), (1, outb)):

    @pl.when(cid == c)
    def _():
      pltpu.sync_copy(acc2.at[pl.ds(row0, ROWS_MAIN)],
                      out.at[pl.ds(row0, ROWS_MAIN)])

      @pl.when(sid == NS - 1)
      def _():
        pltpu.sync_copy(acc2.at[pl.ds(NS * ROWS_MAIN, tail)],
                        out.at[pl.ds(NS * ROWS_MAIN, tail)])


_gnn2 = pl.kernel(
    _gnn_body,
    out_type=(jax.ShapeDtypeStruct((N_NODES, DH), jnp.float32),
              jax.ShapeDtypeStruct((N_NODES, DH), jnp.float32)),
    mesh=_mesh,
    compiler_params=pltpu.CompilerParams(use_tc_tiling_on_sc=False),
    scratch_types=(
        [pltpu.VMEM((CHUNK, DH), jnp.float32)] * 6
        + [pltpu.VMEM((CHUNK,), jnp.int32)] * 9
        + [pltpu.VMEM_SHARED((N_NODES, DH), jnp.float32)] * 2
        + [pltpu.VMEM_SHARED((N_REL, DH), jnp.float32)] * 2
        + [pltpu.SemaphoreType.DMA] * 15
    ),
)


def kernel(node_features, edge_index, etype, rel_emb_0, rel_emb_1):
  src = edge_index[0].astype(jnp.int32)
  dst = edge_index[1].astype(jnp.int32)
  et = etype.astype(jnp.int32)
  ha = node_features[:, :DH]
  hb = node_features[:, DH:]
  oa, ob = _gnn2(ha, hb,
                 rel_emb_0[:, :DH], rel_emb_0[:, DH:],
                 rel_emb_1[:, :DH], rel_emb_1[:, DH:],
                 src, dst, et)
  return jnp.concatenate([oa, ob], axis=1)
